# baseline (device time: 194054 ns/iter reference)
import jax
import jax.numpy as jnp
from jax import lax
from jax.experimental import pallas as pl
from jax.experimental.pallas import tpu as pltpu

N_DEV = 16
NSUB = 4


def kernel(x):
    m_per, n = x.shape
    chunk = m_per // N_DEV
    half = chunk // 2
    sub = half // NSUB
    nst = N_DEV - 1

    def body(x_ref, out_ref, cwbuf, ccwbuf,
             cw_rs_send, cw_rs_recv, ccw_rs_send, ccw_rs_recv,
             cw_ag_send, cw_ag_recv, ccw_ag_send, ccw_ag_recv):
        my = lax.axis_index("i")
        left = lax.rem(my + N_DEV - 1, N_DEV)
        right = lax.rem(my + 1, N_DEV)

        def cidx(k):
            return lax.rem(my + k + 2 * N_DEV, N_DEV)

        def topsub(idx, b):
            return pl.ds(idx * chunk + b * sub, sub)

        def botsub(idx, b):
            return pl.ds(idx * chunk + half + b * sub, sub)

        barrier_sem = pltpu.get_barrier_semaphore()
        pl.semaphore_signal(barrier_sem, 1, device_id=(left,),
                            device_id_type=pl.DeviceIdType.MESH)
        pl.semaphore_signal(barrier_sem, 1, device_id=(right,),
                            device_id_type=pl.DeviceIdType.MESH)
        pl.semaphore_wait(barrier_sem, 2)

        def rs_send(s, b, cw):
            if cw:
                src = (x_ref.at[topsub(cidx(-1), b)] if s == 0
                       else cwbuf.at[s - 1, pl.ds(b * sub, sub)])
                d = pltpu.make_async_remote_copy(
                    src_ref=src,
                    dst_ref=cwbuf.at[s, pl.ds(b * sub, sub)],
                    send_sem=cw_rs_send.at[(NSUB * s + b) % (2 * NSUB)],
                    recv_sem=cw_rs_recv.at[s, b],
                    device_id=(right,), device_id_type=pl.DeviceIdType.MESH)
            else:
                src = (x_ref.at[botsub(cidx(+1), b)] if s == 0
                       else ccwbuf.at[s - 1, pl.ds(b * sub, sub)])
                d = pltpu.make_async_remote_copy(
                    src_ref=src,
                    dst_ref=ccwbuf.at[s, pl.ds(b * sub, sub)],
                    send_sem=ccw_rs_send.at[(NSUB * s + b) % (2 * NSUB)],
                    recv_sem=ccw_rs_recv.at[s, b],
                    device_id=(left,), device_id_type=pl.DeviceIdType.MESH)
            d.start()
            return d

        cw_d = {(0, b): rs_send(0, b, True) for b in range(NSUB)}
        ccw_d = {(0, b): rs_send(0, b, False) for b in range(NSUB)}
        for s in range(nst):
            for b in range(NSUB):
                cw_d[(s, b)].wait_recv()
                cwbuf[s, pl.ds(b * sub, sub)] = (
                    cwbuf[s, pl.ds(b * sub, sub)]
                    + x_ref[topsub(cidx(-2 - s), b), :])
                if s < nst - 1:
                    if s >= 1:
                        cw_d[(s - 1, b)].wait_send()
                    cw_d[(s + 1, b)] = rs_send(s + 1, b, True)
                ccw_d[(s, b)].wait_recv()
                ccwbuf[s, pl.ds(b * sub, sub)] = (
                    ccwbuf[s, pl.ds(b * sub, sub)]
                    + x_ref[botsub(cidx(+2 + s), b), :])
                if s < nst - 1:
                    if s >= 1:
                        ccw_d[(s - 1, b)].wait_send()
                    ccw_d[(s + 1, b)] = rs_send(s + 1, b, False)

        def ag_send(h, b, cw):
            if cw:
                c = cidx(-h)
                src = (cwbuf.at[nst - 1, pl.ds(b * sub, sub)] if h == 0
                       else out_ref.at[topsub(c, b)])
                d = pltpu.make_async_remote_copy(
                    src_ref=src, dst_ref=out_ref.at[topsub(c, b)],
                    send_sem=cw_ag_send.at[(NSUB * h + b) % (2 * NSUB)],
                    recv_sem=cw_ag_recv.at[h, b],
                    device_id=(right,), device_id_type=pl.DeviceIdType.MESH)
            else:
                c = cidx(+h)
                src = (ccwbuf.at[nst - 1, pl.ds(b * sub, sub)] if h == 0
                       else out_ref.at[botsub(c, b)])
                d = pltpu.make_async_remote_copy(
                    src_ref=src, dst_ref=out_ref.at[botsub(c, b)],
                    send_sem=ccw_ag_send.at[(NSUB * h + b) % (2 * NSUB)],
                    recv_sem=ccw_ag_recv.at[h, b],
                    device_id=(left,), device_id_type=pl.DeviceIdType.MESH)
            d.start()
            return d

        cw_a = {(0, b): ag_send(0, b, True) for b in range(NSUB)}
        ccw_a = {(0, b): ag_send(0, b, False) for b in range(NSUB)}
        out_ref[pl.ds(my * chunk, half), :] = cwbuf[nst - 1]
        out_ref[pl.ds(my * chunk + half, half), :] = ccwbuf[nst - 1]

        for h in range(nst):
            for b in range(NSUB):
                cw_a[(h, b)].wait_recv()
                if h < nst - 1:
                    if h >= 1:
                        cw_a[(h - 1, b)].wait_send()
                    cw_a[(h + 1, b)] = ag_send(h + 1, b, True)
                ccw_a[(h, b)].wait_recv()
                if h < nst - 1:
                    if h >= 1:
                        ccw_a[(h - 1, b)].wait_send()
                    ccw_a[(h + 1, b)] = ag_send(h + 1, b, False)

        for dct in (cw_d, ccw_d, cw_a, ccw_a):
            for s in (nst - 2, nst - 1):
                for b in range(NSUB):
                    dct[(s, b)].wait_send()

    return pl.pallas_call(
        body,
        out_shape=jax.ShapeDtypeStruct((m_per, n), x.dtype),
        in_specs=[pl.BlockSpec(memory_space=pltpu.VMEM)],
        out_specs=pl.BlockSpec(memory_space=pltpu.VMEM),
        scratch_shapes=[
            pltpu.VMEM((nst, half, n), x.dtype),
            pltpu.VMEM((nst, half, n), x.dtype),
            pltpu.SemaphoreType.DMA((2 * NSUB,)),
            pltpu.SemaphoreType.DMA((nst, NSUB)),
            pltpu.SemaphoreType.DMA((2 * NSUB,)),
            pltpu.SemaphoreType.DMA((nst, NSUB)),
            pltpu.SemaphoreType.DMA((2 * NSUB,)),
            pltpu.SemaphoreType.DMA((nst, NSUB)),
            pltpu.SemaphoreType.DMA((2 * NSUB,)),
            pltpu.SemaphoreType.DMA((nst, NSUB)),
        ],
        compiler_params=pltpu.CompilerParams(collective_id=0),
    )(x)


# device time: 190585 ns/iter; 1.0182x vs baseline; 1.0182x over previous
import jax
import jax.numpy as jnp
from jax import lax
from jax.experimental import pallas as pl
from jax.experimental.pallas import tpu as pltpu

N_DEV = 16
NSUB = 2

_RING = (0, 1, 2, 3, 7, 6, 5, 9, 10, 11, 15, 14, 13, 12, 8, 4)
_POS = tuple(_RING.index(m) for m in range(N_DEV))
_RIGHT = tuple(_RING[(_POS[m] + 1) % N_DEV] for m in range(N_DEV))
_LEFT = tuple(_RING[(_POS[m] - 1) % N_DEV] for m in range(N_DEV))


def kernel(x):
    m_per, n = x.shape
    chunk = m_per // N_DEV
    half = chunk // 2
    sub = half // NSUB
    nst = N_DEV - 1

    my = lax.axis_index("i")
    info = jnp.stack([
        jnp.asarray(_POS, jnp.int32)[my],
        jnp.asarray(_RIGHT, jnp.int32)[my],
        jnp.asarray(_LEFT, jnp.int32)[my],
    ])

    def body(info_ref, x_ref, out_ref, cwbuf, ccwbuf,
             cw_rs_send, cw_rs_recv, ccw_rs_send, ccw_rs_recv,
             cw_ag_send, cw_ag_recv, ccw_ag_send, ccw_ag_recv):
        pos = info_ref[0]
        right = info_ref[1]
        left = info_ref[2]

        def cidx(k):
            return lax.rem(pos + k + 2 * N_DEV, N_DEV)

        def topsub(idx, b):
            return pl.ds(idx * chunk + b * sub, sub)

        def botsub(idx, b):
            return pl.ds(idx * chunk + half + b * sub, sub)

        barrier_sem = pltpu.get_barrier_semaphore()
        pl.semaphore_signal(barrier_sem, 1, device_id=(left,),
                            device_id_type=pl.DeviceIdType.MESH)
        pl.semaphore_signal(barrier_sem, 1, device_id=(right,),
                            device_id_type=pl.DeviceIdType.MESH)
        pl.semaphore_wait(barrier_sem, 2)

        def rs_send(s, b, cw):
            if cw:
                src = (x_ref.at[topsub(cidx(-1), b)] if s == 0
                       else cwbuf.at[s - 1, pl.ds(b * sub, sub)])
                d = pltpu.make_async_remote_copy(
                    src_ref=src,
                    dst_ref=cwbuf.at[s, pl.ds(b * sub, sub)],
                    send_sem=cw_rs_send.at[(NSUB * s + b) % (2 * NSUB)],
                    recv_sem=cw_rs_recv.at[s, b],
                    device_id=(right,), device_id_type=pl.DeviceIdType.MESH)
            else:
                src = (x_ref.at[botsub(cidx(+1), b)] if s == 0
                       else ccwbuf.at[s - 1, pl.ds(b * sub, sub)])
                d = pltpu.make_async_remote_copy(
                    src_ref=src,
                    dst_ref=ccwbuf.at[s, pl.ds(b * sub, sub)],
                    send_sem=ccw_rs_send.at[(NSUB * s + b) % (2 * NSUB)],
                    recv_sem=ccw_rs_recv.at[s, b],
                    device_id=(left,), device_id_type=pl.DeviceIdType.MESH)
            d.start()
            return d

        def ag_send(h, b, cw):
            if cw:
                c = cidx(-h)
                src = (cwbuf.at[nst - 1, pl.ds(b * sub, sub)] if h == 0
                       else out_ref.at[topsub(c, b)])
                d = pltpu.make_async_remote_copy(
                    src_ref=src, dst_ref=out_ref.at[topsub(c, b)],
                    send_sem=cw_ag_send.at[(NSUB * h + b) % (2 * NSUB)],
                    recv_sem=cw_ag_recv.at[h, b],
                    device_id=(right,), device_id_type=pl.DeviceIdType.MESH)
            else:
                c = cidx(+h)
                src = (ccwbuf.at[nst - 1, pl.ds(b * sub, sub)] if h == 0
                       else out_ref.at[botsub(c, b)])
                d = pltpu.make_async_remote_copy(
                    src_ref=src, dst_ref=out_ref.at[botsub(c, b)],
                    send_sem=ccw_ag_send.at[(NSUB * h + b) % (2 * NSUB)],
                    recv_sem=ccw_ag_recv.at[h, b],
                    device_id=(left,), device_id_type=pl.DeviceIdType.MESH)
            d.start()
            return d

        cw_d = {(0, b): rs_send(0, b, True) for b in range(NSUB)}
        ccw_d = {(0, b): rs_send(0, b, False) for b in range(NSUB)}
        cw_a = {}
        ccw_a = {}
        for s in range(nst):
            for b in range(NSUB):
                cw_d[(s, b)].wait_recv()
                cwbuf[s, pl.ds(b * sub, sub)] = (
                    cwbuf[s, pl.ds(b * sub, sub)]
                    + x_ref[topsub(cidx(-2 - s), b), :])
                if s < nst - 1:
                    if s >= 1:
                        cw_d[(s - 1, b)].wait_send()
                    cw_d[(s + 1, b)] = rs_send(s + 1, b, True)
                else:
                    cw_a[(0, b)] = ag_send(0, b, True)
                ccw_d[(s, b)].wait_recv()
                ccwbuf[s, pl.ds(b * sub, sub)] = (
                    ccwbuf[s, pl.ds(b * sub, sub)]
                    + x_ref[botsub(cidx(+2 + s), b), :])
                if s < nst - 1:
                    if s >= 1:
                        ccw_d[(s - 1, b)].wait_send()
                    ccw_d[(s + 1, b)] = rs_send(s + 1, b, False)
                else:
                    ccw_a[(0, b)] = ag_send(0, b, False)

        out_ref[pl.ds(pos * chunk, half), :] = cwbuf[nst - 1]
        out_ref[pl.ds(pos * chunk + half, half), :] = ccwbuf[nst - 1]

        for h in range(nst):
            for b in range(NSUB):
                cw_a[(h, b)].wait_recv()
                if h < nst - 1:
                    if h >= 1:
                        cw_a[(h - 1, b)].wait_send()
                    cw_a[(h + 1, b)] = ag_send(h + 1, b, True)
                ccw_a[(h, b)].wait_recv()
                if h < nst - 1:
                    if h >= 1:
                        ccw_a[(h - 1, b)].wait_send()
                    ccw_a[(h + 1, b)] = ag_send(h + 1, b, False)

        for dct in (cw_d, ccw_d, cw_a, ccw_a):
            for s in (nst - 2, nst - 1):
                for b in range(NSUB):
                    dct[(s, b)].wait_send()

    return pl.pallas_call(
        body,
        out_shape=jax.ShapeDtypeStruct((m_per, n), x.dtype),
        in_specs=[
            pl.BlockSpec(memory_space=pltpu.SMEM),
            pl.BlockSpec(memory_space=pltpu.VMEM),
        ],
        out_specs=pl.BlockSpec(memory_space=pltpu.VMEM),
        scratch_shapes=[
            pltpu.VMEM((nst, half, n), x.dtype),
            pltpu.VMEM((nst, half, n), x.dtype),
            pltpu.SemaphoreType.DMA((2 * NSUB,)),
            pltpu.SemaphoreType.DMA((nst, NSUB)),
            pltpu.SemaphoreType.DMA((2 * NSUB,)),
            pltpu.SemaphoreType.DMA((nst, NSUB)),
            pltpu.SemaphoreType.DMA((2 * NSUB,)),
            pltpu.SemaphoreType.DMA((nst, NSUB)),
            pltpu.SemaphoreType.DMA((2 * NSUB,)),
            pltpu.SemaphoreType.DMA((nst, NSUB)),
        ],
        compiler_params=pltpu.CompilerParams(collective_id=0),
    )(info, x)


# device time: 185858 ns/iter; 1.0441x vs baseline; 1.0254x over previous
import jax
import jax.numpy as jnp
from jax import lax
from jax.experimental import pallas as pl
from jax.experimental.pallas import tpu as pltpu

N_DEV = 16
NSUB = 2

_RING = (0, 1, 2, 3, 7, 6, 5, 9, 10, 11, 15, 14, 13, 12, 8, 4)
_POS = tuple(_RING.index(m) for m in range(N_DEV))
_RIGHT = tuple(_RING[(_POS[m] + 1) % N_DEV] for m in range(N_DEV))
_LEFT = tuple(_RING[(_POS[m] - 1) % N_DEV] for m in range(N_DEV))


def kernel(x):
    m_per, n = x.shape
    chunk = m_per // N_DEV
    half = chunk // 2
    sub = half // NSUB
    nst = N_DEV - 1

    def body(x_ref, out_ref, cwbuf, ccwbuf,
             cw_rs_send, cw_rs_recv, ccw_rs_send, ccw_rs_recv,
             cw_ag_send, cw_ag_recv, ccw_ag_send, ccw_ag_recv):
        my = lax.axis_index("i")

        def lut(table):
            v = jnp.int32(table[0])
            for m in range(1, N_DEV):
                v = jnp.where(my == m, jnp.int32(table[m]), v)
            return v

        pos = lut(_POS)
        right = lut(_RIGHT)
        left = lut(_LEFT)

        def cidx(k):
            return lax.rem(pos + k + 2 * N_DEV, N_DEV)

        def topsub(idx, b):
            return pl.ds(idx * chunk + b * sub, sub)

        def botsub(idx, b):
            return pl.ds(idx * chunk + half + b * sub, sub)

        barrier_sem = pltpu.get_barrier_semaphore()
        pl.semaphore_signal(barrier_sem, 1, device_id=(left,),
                            device_id_type=pl.DeviceIdType.MESH)
        pl.semaphore_signal(barrier_sem, 1, device_id=(right,),
                            device_id_type=pl.DeviceIdType.MESH)
        pl.semaphore_wait(barrier_sem, 2)

        def rs_send(s, b, cw):
            if cw:
                src = (x_ref.at[topsub(cidx(-1), b)] if s == 0
                       else cwbuf.at[s - 1, pl.ds(b * sub, sub)])
                d = pltpu.make_async_remote_copy(
                    src_ref=src,
                    dst_ref=cwbuf.at[s, pl.ds(b * sub, sub)],
                    send_sem=cw_rs_send.at[(NSUB * s + b) % (2 * NSUB)],
                    recv_sem=cw_rs_recv.at[s, b],
                    device_id=(right,), device_id_type=pl.DeviceIdType.MESH)
            else:
                src = (x_ref.at[botsub(cidx(+1), b)] if s == 0
                       else ccwbuf.at[s - 1, pl.ds(b * sub, sub)])
                d = pltpu.make_async_remote_copy(
                    src_ref=src,
                    dst_ref=ccwbuf.at[s, pl.ds(b * sub, sub)],
                    send_sem=ccw_rs_send.at[(NSUB * s + b) % (2 * NSUB)],
                    recv_sem=ccw_rs_recv.at[s, b],
                    device_id=(left,), device_id_type=pl.DeviceIdType.MESH)
            d.start()
            return d

        def ag_send(h, b, cw):
            if cw:
                c = cidx(-h)
                src = (cwbuf.at[nst - 1, pl.ds(b * sub, sub)] if h == 0
                       else out_ref.at[topsub(c, b)])
                d = pltpu.make_async_remote_copy(
                    src_ref=src, dst_ref=out_ref.at[topsub(c, b)],
                    send_sem=cw_ag_send.at[(NSUB * h + b) % (2 * NSUB)],
                    recv_sem=cw_ag_recv.at[h, b],
                    device_id=(right,), device_id_type=pl.DeviceIdType.MESH)
            else:
                c = cidx(+h)
                src = (ccwbuf.at[nst - 1, pl.ds(b * sub, sub)] if h == 0
                       else out_ref.at[botsub(c, b)])
                d = pltpu.make_async_remote_copy(
                    src_ref=src, dst_ref=out_ref.at[botsub(c, b)],
                    send_sem=ccw_ag_send.at[(NSUB * h + b) % (2 * NSUB)],
                    recv_sem=ccw_ag_recv.at[h, b],
                    device_id=(left,), device_id_type=pl.DeviceIdType.MESH)
            d.start()
            return d

        cw_d = {(0, b): rs_send(0, b, True) for b in range(NSUB)}
        ccw_d = {(0, b): rs_send(0, b, False) for b in range(NSUB)}
        cw_a = {}
        ccw_a = {}
        for s in range(nst):
            for b in range(NSUB):
                cw_d[(s, b)].wait_recv()
                cwbuf[s, pl.ds(b * sub, sub)] = (
                    cwbuf[s, pl.ds(b * sub, sub)]
                    + x_ref[topsub(cidx(-2 - s), b), :])
                if s < nst - 1:
                    if s >= 1:
                        cw_d[(s - 1, b)].wait_send()
                    cw_d[(s + 1, b)] = rs_send(s + 1, b, True)
                else:
                    cw_a[(0, b)] = ag_send(0, b, True)
                ccw_d[(s, b)].wait_recv()
                ccwbuf[s, pl.ds(b * sub, sub)] = (
                    ccwbuf[s, pl.ds(b * sub, sub)]
                    + x_ref[botsub(cidx(+2 + s), b), :])
                if s < nst - 1:
                    if s >= 1:
                        ccw_d[(s - 1, b)].wait_send()
                    ccw_d[(s + 1, b)] = rs_send(s + 1, b, False)
                else:
                    ccw_a[(0, b)] = ag_send(0, b, False)

        out_ref[pl.ds(pos * chunk, half), :] = cwbuf[nst - 1]
        out_ref[pl.ds(pos * chunk + half, half), :] = ccwbuf[nst - 1]

        for h in range(nst):
            for b in range(NSUB):
                cw_a[(h, b)].wait_recv()
                if h < nst - 1:
                    if h >= 1:
                        cw_a[(h - 1, b)].wait_send()
                    cw_a[(h + 1, b)] = ag_send(h + 1, b, True)
                ccw_a[(h, b)].wait_recv()
                if h < nst - 1:
                    if h >= 1:
                        ccw_a[(h - 1, b)].wait_send()
                    ccw_a[(h + 1, b)] = ag_send(h + 1, b, False)

        for dct in (cw_d, ccw_d, cw_a, ccw_a):
            for s in (nst - 2, nst - 1):
                for b in range(NSUB):
                    dct[(s, b)].wait_send()

    return pl.pallas_call(
        body,
        out_shape=jax.ShapeDtypeStruct((m_per, n), x.dtype),
        in_specs=[pl.BlockSpec(memory_space=pltpu.VMEM)],
        out_specs=pl.BlockSpec(memory_space=pltpu.VMEM),
        scratch_shapes=[
            pltpu.VMEM((nst, half, n), x.dtype),
            pltpu.VMEM((nst, half, n), x.dtype),
            pltpu.SemaphoreType.DMA((2 * NSUB,)),
            pltpu.SemaphoreType.DMA((nst, NSUB)),
            pltpu.SemaphoreType.DMA((2 * NSUB,)),
            pltpu.SemaphoreType.DMA((nst, NSUB)),
            pltpu.SemaphoreType.DMA((2 * NSUB,)),
            pltpu.SemaphoreType.DMA((nst, NSUB)),
            pltpu.SemaphoreType.DMA((2 * NSUB,)),
            pltpu.SemaphoreType.DMA((nst, NSUB)),
        ],
        compiler_params=pltpu.CompilerParams(collective_id=0),
    )(x)
